# baseline (speedup 1.0000x reference)
import functools

import jax
import jax.numpy as jnp
import numpy as np
from jax import lax
from jax.experimental import pallas as pl
from jax.experimental.pallas import tpu as pltpu
from jax.experimental.pallas import tpu_sc as plsc

N_NODES = 10000
N_EDGES = 320000
HEADS = 8
DH = 8
HID = 64

NC = 2
NS = 16
NTILES = NC * NS
EDGES_PER_TILE = N_EDGES // NTILES
EDGE_BATCH = 40
N_BATCHES = EDGES_PER_TILE // EDGE_BATCH
N_PAD = 10240
ROWS_PER_TILE = N_PAD // NS

_P = np.empty(64, np.int32)
for _h in range(HEADS):
    for _d in range(DH):
        _col = 16 * (_d // 2) + (_h if _d % 2 == 0 else 15 - _h)
        _P[_col] = _h * DH + _d


def _edge_body(qt, kvt, et, src, dst, zac,
               out_acc,
               acc, srcb, dstb, kvb, qb, eb, ob,
               semA, semB, semI0, semI1):
    cid = lax.axis_index("c")
    sid = lax.axis_index("s")
    wid = cid * NS + sid

    r0 = sid * ROWS_PER_TILE
    pltpu.sync_copy(zac.at[pl.ds(r0, ROWS_PER_TILE)],
                    acc.at[pl.ds(r0, ROWS_PER_TILE)])

    def zrow(i, c):
        for j in range(5, 8):
            ob[i, pl.ds(j * 16, 16)] = jnp.zeros((16,), jnp.float32)
        return c

    lax.fori_loop(0, EDGE_BATCH, zrow, 0)
    plsc.subcore_barrier()

    base0 = wid * EDGES_PER_TILE

    def fire_idx_src(b, buf, semI):
        base = base0 + b * EDGE_BATCH
        pltpu.async_copy(src.at[pl.ds(base, EDGE_BATCH)], srcb.at[buf], semI)

    def fire_idx_dst(b, buf, semI):
        base = base0 + b * EDGE_BATCH
        pltpu.async_copy(dst.at[pl.ds(base, EDGE_BATCH)], dstb.at[buf], semI)

    def fire_gathers(b, buf, sem, semI):
        base = base0 + b * EDGE_BATCH
        pltpu.make_async_copy(src.at[pl.ds(0, EDGE_BATCH)], srcb.at[buf],
                              semI).wait()
        pltpu.make_async_copy(dst.at[pl.ds(0, EDGE_BATCH)], dstb.at[buf],
                              semI).wait()
        pltpu.async_copy(kvt.at[srcb.at[buf]], kvb.at[buf], sem)
        pltpu.async_copy(qt.at[dstb.at[buf]], qb.at[buf], sem)
        pltpu.async_copy(et.at[pl.ds(base, EDGE_BATCH)], eb.at[buf], sem)

    def drain_gathers(buf, sem):
        pltpu.make_async_copy(kvt.at[pl.ds(0, EDGE_BATCH)], kvb.at[buf],
                              sem).wait()
        pltpu.make_async_copy(qt.at[pl.ds(0, EDGE_BATCH)], qb.at[buf],
                              sem).wait()
        pltpu.make_async_copy(et.at[pl.ds(0, EDGE_BATCH)], eb.at[buf],
                              sem).wait()

    def compute(b, buf, semI, nb2_ok):
        def edge(i, carry2):
            t = None
            for j in range(4):
                sl = pl.ds(j * 16, 16)
                kq = kvb[buf, i, sl] * qb[buf, i, sl] * eb[buf, i, sl]
                t = kq if t is None else t + kq
            s = t + lax.rev(t, (0,))
            s = jnp.minimum(jnp.maximum(s, -5.0), 5.0)
            w = jnp.exp(s)
            for j in range(4):
                ob[i, pl.ds(j * 16, 16)] = kvb[buf, i, pl.ds(64 + j * 16, 16)] * w
            ob[i, pl.ds(64, 16)] = w
            return carry2

        lax.fori_loop(0, EDGE_BATCH, edge, 0)
        pltpu.sync_copy(ob, acc.at[dstb.at[buf]], add=True)

        @pl.when(nb2_ok)
        def _():
            fire_idx_dst(b + 2, buf, semI)

    fire_idx_src(0, 0, semI0)
    fire_idx_dst(0, 0, semI0)
    fire_idx_src(1, 1, semI1)
    fire_idx_dst(1, 1, semI1)
    fire_gathers(0, 0, semA, semI0)

    def pair(g, carry):
        b0 = 2 * g

        @pl.when(b0 + 1 < N_BATCHES)
        def _():
            fire_gathers(b0 + 1, 1, semB, semI1)

        drain_gathers(0, semA)

        @pl.when(b0 + 2 < N_BATCHES)
        def _():
            fire_idx_src(b0 + 2, 0, semI0)

        compute(b0, 0, semI0, b0 + 2 < N_BATCHES)

        @pl.when(b0 + 2 < N_BATCHES)
        def _():
            fire_gathers(b0 + 2, 0, semA, semI0)

        @pl.when(b0 + 1 < N_BATCHES)
        def _():
            drain_gathers(1, semB)

            @pl.when(b0 + 3 < N_BATCHES)
            def _():
                fire_idx_src(b0 + 3, 1, semI1)

            compute(b0 + 1, 1, semI1, b0 + 3 < N_BATCHES)

        return carry

    lax.fori_loop(0, (N_BATCHES + 1) // 2, pair, 0)
    plsc.subcore_barrier()

    pltpu.sync_copy(acc.at[pl.ds(r0, ROWS_PER_TILE)],
                    out_acc.at[cid, pl.ds(r0, ROWS_PER_TILE)])


@functools.cache
def _edge_call():
    return pl.kernel(
        _edge_body,
        out_type=jax.ShapeDtypeStruct((NC, N_PAD, 128), jnp.float32),
        mesh=plsc.VectorSubcoreMesh(core_axis_name="c", subcore_axis_name="s"),
        scratch_types=[
            pltpu.VMEM_SHARED((N_PAD, 128), jnp.float32),
            pltpu.VMEM((2, EDGE_BATCH), jnp.int32),
            pltpu.VMEM((2, EDGE_BATCH), jnp.int32),
            pltpu.VMEM((2, EDGE_BATCH, 128), jnp.float32),
            pltpu.VMEM((2, EDGE_BATCH, 128), jnp.float32),
            pltpu.VMEM((2, EDGE_BATCH, 64), jnp.float32),
            pltpu.VMEM((EDGE_BATCH, 128), jnp.float32),
            pltpu.SemaphoreType.DMA,
            pltpu.SemaphoreType.DMA,
            pltpu.SemaphoreType.DMA,
            pltpu.SemaphoreType.DMA,
        ],
    )


def _edge_phase(qt, kvt, et, src, dst):
    zac = jnp.zeros((N_PAD, 128), jnp.float32)
    return _edge_call()(qt, kvt, et, src, dst, zac)


def _dot(a, b):
    return jnp.dot(a, b, preferred_element_type=jnp.float32)


def _bn_fwd(x, g, b):
    m = jnp.mean(x, axis=0, keepdims=True)
    v = jnp.mean((x - m) ** 2, axis=0, keepdims=True)
    return g * (x - m) * jax.lax.rsqrt(v + 1e-5) + b


def _prep_body(h_r, wh_r, bh_r, wq_r, wkv_r, hh_o, qt_o, kvt_o):
    hh = _dot(h_r[...], wh_r[...]) + bh_r[...]
    hh_o[...] = hh
    q = _dot(hh, wq_r[...])
    qt_o[...] = jnp.concatenate([q, jnp.zeros_like(q)], axis=1)
    kvt_o[...] = _dot(hh, wkv_r[...])


def _prep(h, wh, bh, wq_t, wkv_t):
    f = jax.ShapeDtypeStruct
    return pl.pallas_call(
        _prep_body,
        out_shape=(
            f((N_NODES, HID), jnp.float32),
            f((N_NODES, 128), jnp.float32),
            f((N_NODES, 128), jnp.float32),
        ),
    )(h, wh, bh, wq_t, wkv_t)


_EBLK = 3200


def _eproj_body(e_r, m_r, c_r, et_o):
    et_o[...] = _dot(e_r[...], m_r[...]) + c_r[...]


def _eproj(e, m, c):
    return pl.pallas_call(
        _eproj_body,
        grid=(N_EDGES // _EBLK,),
        in_specs=[
            pl.BlockSpec((_EBLK, 16), lambda i: (i, 0)),
            pl.BlockSpec((16, 64), lambda i: (0, 0)),
            pl.BlockSpec((1, 64), lambda i: (0, 0)),
        ],
        out_specs=pl.BlockSpec((_EBLK, 64), lambda i: (i, 0)),
        out_shape=jax.ShapeDtypeStruct((N_EDGES, 64), jnp.float32),
    )(e, m, c)


def _attn_out(part, wo_p, bo):
    p = (part[0] + part[1])[:N_NODES]
    wv = p[:, 0:64]
    z16 = p[:, 64:80]
    zc = jnp.concatenate([z16, z16, z16, z16], axis=1)
    ha = wv / (zc + 1e-6)
    return _dot(ha, wo_p) + bo


def _ffn(x, w1, b1, w2, b2):
    t = jax.nn.relu(_dot(x, w1) + b1)
    return _dot(t, w2) + b2


def _post0_body(part_r, hh0_r, wo_r, bo_r, g1_r, be1_r, w1_r, b1_r,
                w2_r, b2_r, g2_r, be2_r, wq_r, wkv_r,
                qt_o, kvt_o):
    hh = _attn_out(part_r[...], wo_r[...], bo_r[...])
    hh = hh0_r[...] + hh
    hh = _bn_fwd(hh, g1_r[...], be1_r[...])
    hh = hh + _ffn(hh, w1_r[...], b1_r[...], w2_r[...], b2_r[...])
    hh = _bn_fwd(hh, g2_r[...], be2_r[...])
    q = _dot(hh, wq_r[...])
    qt_o[...] = jnp.concatenate([q, jnp.zeros_like(q)], axis=1)
    kvt_o[...] = _dot(hh, wkv_r[...])


def _post0(part, hh0, wo_p, bo, g1, be1, w1, b1, w2, b2, g2, be2,
           wq1_t, wkv1_t):
    f = jax.ShapeDtypeStruct
    return pl.pallas_call(
        _post0_body,
        out_shape=(
            f((N_NODES, 128), jnp.float32),
            f((N_NODES, 128), jnp.float32),
        ),
    )(part, hh0, wo_p, bo, g1, be1, w1, b1, w2, b2, g2, be2,
      wq1_t, wkv1_t)


def _post1_body(part_r, wo_r, bo_r, g1_r, be1_r, w1_r, b1_r,
                w2_r, b2_r, g2_r, be2_r, mw0_r, mb0_r, mw1_r, mb1_r,
                mw2_r, mb2_r, y_o):
    hh = _attn_out(part_r[...], wo_r[...], bo_r[...])
    hh = _bn_fwd(hh, g1_r[...], be1_r[...])
    hh = _ffn(hh, w1_r[...], b1_r[...], w2_r[...], b2_r[...])
    hh = _bn_fwd(hh, g2_r[...], be2_r[...])
    hg = jnp.mean(hh, axis=0, keepdims=True)
    y = jax.nn.relu(_dot(hg, mw0_r[...]) + mb0_r[...])
    y = jax.nn.relu(_dot(y, mw1_r[...]) + mb1_r[...])
    y_o[...] = _dot(y, mw2_r[...]) + mb2_r[...]


def _post1(part, wo_p, bo, g1, be1, w1, b1, w2, b2, g2, be2,
           mw0, mb0, mw1, mb1, mw2, mb2):
    return pl.pallas_call(
        _post1_body,
        out_shape=jax.ShapeDtypeStruct((1, 3), jnp.float32),
    )(part, wo_p, bo, g1, be1, w1, b1, w2, b2, g2, be2,
      mw0, mb0, mw1, mb1, mw2, mb2)


def _row(v):
    return v.reshape(1, -1)


def kernel(h, e, edge_index, params):
    p0 = params['layer0']
    p1 = params['layer1']
    scale = np.float32(1.0 / np.sqrt(DH))

    wq0 = p0['Wq'][:, _P] * scale
    wk0 = p0['Wk'][:, _P]
    wv0 = p0['Wv'][:, _P]
    wq1 = p1['Wq'][:, _P] * scale
    wk1 = p1['Wk'][:, _P]
    wv1 = p1['Wv'][:, _P]
    wo0 = p0['Wo'][_P, :]
    wo1 = p1['Wo'][_P, :]
    m0 = (params['We_emb'] @ p0['We'])[:, _P]
    c0 = _row((params['be_emb'] @ p0['We'])[_P])
    m1 = (params['We_emb'] @ p1['We'])[:, _P]
    c1 = _row((params['be_emb'] @ p1['We'])[_P])

    src = edge_index[0]
    dst = edge_index[1]

    wkv0 = jnp.concatenate([wk0, wv0], axis=1)
    wkv1 = jnp.concatenate([wk1, wv1], axis=1)

    hh0, qt0, kvt0 = _prep(h, params['Wh'], _row(params['bh']), wq0, wkv0)
    et0 = _eproj(e, m0, c0)

    part0 = _edge_phase(qt0, kvt0, et0, src, dst)
    et1 = _eproj(e, m1, c1)
    qt1, kvt1 = _post0(
        part0, hh0, wo0, _row(p0['bo']),
        _row(p0['bn1_g']), _row(p0['bn1_b']),
        p0['W1'], _row(p0['b1']), p0['W2'], _row(p0['b2']),
        _row(p0['bn2_g']), _row(p0['bn2_b']),
        wq1, wkv1)

    part1 = _edge_phase(qt1, kvt1, et1, src, dst)
    y = _post1(
        part1, wo1, _row(p1['bo']),
        _row(p1['bn1_g']), _row(p1['bn1_b']),
        p1['W1'], _row(p1['b1']), p1['W2'], _row(p1['b2']),
        _row(p1['bn2_g']), _row(p1['bn2_b']),
        params['mlp_W0'], _row(params['mlp_b0']),
        params['mlp_W1'], _row(params['mlp_b1']),
        params['mlp_W2'], _row(params['mlp_b2']))
    return y
